# baseline (device time: 20722 ns/iter reference)
import jax
import jax.numpy as jnp
from jax import lax
from jax.experimental import pallas as pl
from jax.experimental.pallas import tpu as pltpu

Q_ROWS = 256
SUB = 32
NQ = Q_ROWS // SUB
N_X = NQ + NQ // 2
XQ_ORDER = list(range(NQ // 2, NQ)) + list(range(NQ // 2))
Y_RELAY = tuple(range(NQ // 2, 3 * NQ // 4))
Z_RELAY = tuple(range(3 * NQ // 4, NQ))


def kernel(x, pi):
    shard_shape = x.shape
    ncols = shard_shape[2]

    def body(x_ref, pi_ref, out_ref, stage_ref, commx_ref, gath_ref,
             copy_sem, out_sem, xs_sem, xr_sem, ys_sem, yr_sem,
             zs_sem, zr_sem):
        my_x = lax.axis_index("x")
        my_y = lax.axis_index("y")
        my_z = lax.axis_index("z")
        tgt_x = pi_ref[my_x]
        dev_x = (tgt_x, my_y, my_z)
        dev_y = (my_x, 1 - my_y, my_z)
        dev_z = (my_x, my_y, 1 - my_z)

        q_me = 2 * my_y + my_z
        q_diag = 2 * (1 - my_y) + (1 - my_z)
        q_yn = 2 * (1 - my_y) + my_z
        q_zn = 2 * my_y + (1 - my_z)

        def xrow(k):
            if k < NQ:
                return q_me * Q_ROWS + XQ_ORDER[k] * SUB
            return q_diag * Q_ROWS + (k - NQ) * SUB

        barrier = pltpu.get_barrier_semaphore()
        for dev in (dev_x, dev_y, dev_z):
            pl.semaphore_signal(
                barrier, inc=1,
                device_id=dev, device_id_type=pl.DeviceIdType.MESH,
            )

        def start_fetch(k):
            cp = pltpu.make_async_copy(
                x_ref.at[0, pl.ds(xrow(k), SUB), :],
                stage_ref.at[k % 2],
                copy_sem.at[k % 2],
            )
            cp.start()
            return cp

        copyouts = []

        def copyout(row):
            cp = pltpu.make_async_copy(
                gath_ref.at[pl.ds(row, SUB), :],
                out_ref.at[0, pl.ds(row, SUB), :],
                out_sem.at[len(copyouts)],
            )
            cp.start()
            copyouts.append(cp)

        fetches = {0: start_fetch(0)}
        x_rdmas = []
        for k in range(N_X):
            if k + 1 < N_X:
                fetches[k + 1] = start_fetch(k + 1)
            fetches[k].wait()
            commx_ref[pl.ds(k * SUB, SUB), :] = (
                stage_ref[k % 2].astype(jnp.bfloat16))
            if k == 0:
                pl.semaphore_wait(barrier, 3)
            rdma = pltpu.make_async_remote_copy(
                src_ref=commx_ref.at[pl.ds(k * SUB, SUB), :],
                dst_ref=gath_ref.at[pl.ds(xrow(k), SUB), :],
                send_sem=xs_sem.at[k],
                recv_sem=xr_sem.at[k],
                device_id=dev_x,
                device_id_type=pl.DeviceIdType.MESH,
            )
            rdma.start()
            x_rdmas.append(rdma)

        def fwd(row, ssem, rsem, dev):
            r = pltpu.make_async_remote_copy(
                src_ref=gath_ref.at[pl.ds(row, SUB), :],
                dst_ref=gath_ref.at[pl.ds(row, SUB), :],
                send_sem=ssem,
                recv_sem=rsem,
                device_id=dev,
                device_id_type=pl.DeviceIdType.MESH,
            )
            r.start()
            return r

        y_rdmas = []
        z_rdmas = []
        y_recv_waited = set()
        z_recv_waited = set()
        for k in range(NQ):
            x_rdmas[k].wait_recv()
            row = q_me * Q_ROWS + XQ_ORDER[k] * SUB
            y_rdmas.append(fwd(row, ys_sem.at[k], yr_sem.at[k], dev_y))
            z_rdmas.append(fwd(row, zs_sem.at[k], zr_sem.at[k], dev_z))
            copyout(row)

        relay_rdmas = []
        for i, c in enumerate(Y_RELAY):
            pos = XQ_ORDER.index(c)
            z_rdmas[pos].wait_recv()
            z_recv_waited.add(pos)
            row = q_zn * Q_ROWS + c * SUB
            relay_rdmas.append(
                fwd(row, ys_sem.at[NQ + i], yr_sem.at[NQ + i], dev_y))
            copyout(row)
        for i, c in enumerate(Z_RELAY):
            pos = XQ_ORDER.index(c)
            y_rdmas[pos].wait_recv()
            y_recv_waited.add(pos)
            row = q_yn * Q_ROWS + c * SUB
            relay_rdmas.append(
                fwd(row, zs_sem.at[NQ + i], zr_sem.at[NQ + i], dev_z))
            copyout(row)

        for k in range(N_X):
            x_rdmas[k].wait_send()
            if k >= NQ:
                x_rdmas[k].wait_recv()
                copyout(xrow(k))
        for k in range(NQ):
            y_rdmas[k].wait_send()
            if k not in y_recv_waited:
                y_rdmas[k].wait_recv()
                copyout(q_yn * Q_ROWS + XQ_ORDER[k] * SUB)
            z_rdmas[k].wait_send()
            if k not in z_recv_waited:
                z_rdmas[k].wait_recv()
                copyout(q_zn * Q_ROWS + XQ_ORDER[k] * SUB)
        for i, c in enumerate(Y_RELAY):
            relay_rdmas[i].wait()
            copyout(q_diag * Q_ROWS + c * SUB)
        for i, c in enumerate(Z_RELAY):
            relay_rdmas[len(Y_RELAY) + i].wait()
            copyout(q_diag * Q_ROWS + c * SUB)
        for cp in copyouts:
            cp.wait()

    return pl.pallas_call(
        body,
        out_shape=jax.ShapeDtypeStruct(shard_shape, jnp.bfloat16),
        in_specs=[
            pl.BlockSpec(memory_space=pl.ANY),
            pl.BlockSpec(memory_space=pltpu.SMEM),
        ],
        out_specs=pl.BlockSpec(memory_space=pl.ANY),
        scratch_shapes=[
            pltpu.VMEM((2, SUB, ncols), x.dtype),
            pltpu.VMEM((N_X * SUB, ncols), jnp.bfloat16),
            pltpu.VMEM((4 * Q_ROWS, ncols), jnp.bfloat16),
            pltpu.SemaphoreType.DMA((2,)),
            pltpu.SemaphoreType.DMA((4 * NQ,)),
            pltpu.SemaphoreType.DMA((N_X,)),
            pltpu.SemaphoreType.DMA((N_X,)),
            pltpu.SemaphoreType.DMA((NQ + len(Y_RELAY),)),
            pltpu.SemaphoreType.DMA((NQ + len(Y_RELAY),)),
            pltpu.SemaphoreType.DMA((NQ + len(Z_RELAY),)),
            pltpu.SemaphoreType.DMA((NQ + len(Z_RELAY),)),
        ],
        compiler_params=pltpu.CompilerParams(collective_id=0),
    )(x, pi)


# device time: 20122 ns/iter; 1.0298x vs baseline; 1.0298x over previous
import jax
import jax.numpy as jnp
from jax import lax
from jax.experimental import pallas as pl
from jax.experimental.pallas import tpu as pltpu

Q_ROWS = 256
SUB = 32
NQ = Q_ROWS // SUB
N_X = NQ + NQ // 2
XQ_ORDER = list(range(NQ // 2, NQ)) + list(range(NQ // 2))
Y_RELAY = tuple(range(NQ // 2, 3 * NQ // 4))
Z_RELAY = tuple(range(3 * NQ // 4, NQ))


def kernel(x, pi):
    shard_shape = x.shape
    ncols = shard_shape[2]

    def body(x_ref, pi_ref, out_ref, stage_ref, commx_ref,
             copy_sem, xs_sem, xr_sem, ys_sem, yr_sem, zs_sem, zr_sem):
        my_x = lax.axis_index("x")
        my_y = lax.axis_index("y")
        my_z = lax.axis_index("z")
        tgt_x = pi_ref[my_x]
        dev_x = (tgt_x, my_y, my_z)
        dev_y = (my_x, 1 - my_y, my_z)
        dev_z = (my_x, my_y, 1 - my_z)

        q_me = 2 * my_y + my_z
        q_diag = 2 * (1 - my_y) + (1 - my_z)
        q_yn = 2 * (1 - my_y) + my_z
        q_zn = 2 * my_y + (1 - my_z)

        def xrow(k):
            if k < NQ:
                return q_me * Q_ROWS + XQ_ORDER[k] * SUB
            return q_diag * Q_ROWS + (k - NQ) * SUB

        barrier = pltpu.get_barrier_semaphore()
        for dev in (dev_x, dev_y, dev_z):
            pl.semaphore_signal(
                barrier, inc=1,
                device_id=dev, device_id_type=pl.DeviceIdType.MESH,
            )

        def start_fetch(k):
            cp = pltpu.make_async_copy(
                x_ref.at[0, pl.ds(xrow(k), SUB), :],
                stage_ref.at[k % 2],
                copy_sem.at[k % 2],
            )
            cp.start()
            return cp

        fetches = {0: start_fetch(0)}
        x_rdmas = []
        for k in range(N_X):
            if k + 1 < N_X:
                fetches[k + 1] = start_fetch(k + 1)
            fetches[k].wait()
            commx_ref[pl.ds(k * SUB, SUB), :] = (
                stage_ref[k % 2].astype(jnp.bfloat16))
            if k == 0:
                pl.semaphore_wait(barrier, 3)
            rdma = pltpu.make_async_remote_copy(
                src_ref=commx_ref.at[pl.ds(k * SUB, SUB), :],
                dst_ref=out_ref.at[0, pl.ds(xrow(k), SUB), :],
                send_sem=xs_sem.at[k],
                recv_sem=xr_sem.at[k],
                device_id=dev_x,
                device_id_type=pl.DeviceIdType.MESH,
            )
            rdma.start()
            x_rdmas.append(rdma)

        def fwd(row, ssem, rsem, dev):
            r = pltpu.make_async_remote_copy(
                src_ref=out_ref.at[0, pl.ds(row, SUB), :],
                dst_ref=out_ref.at[0, pl.ds(row, SUB), :],
                send_sem=ssem,
                recv_sem=rsem,
                device_id=dev,
                device_id_type=pl.DeviceIdType.MESH,
            )
            r.start()
            return r

        y_rdmas = []
        z_rdmas = []
        y_recv_waited = set()
        z_recv_waited = set()
        for k in range(NQ):
            x_rdmas[k].wait_recv()
            row = q_me * Q_ROWS + XQ_ORDER[k] * SUB
            y_rdmas.append(fwd(row, ys_sem.at[k], yr_sem.at[k], dev_y))
            z_rdmas.append(fwd(row, zs_sem.at[k], zr_sem.at[k], dev_z))

        relay_rdmas = []
        for i, c in enumerate(Y_RELAY):
            pos = XQ_ORDER.index(c)
            z_rdmas[pos].wait_recv()
            z_recv_waited.add(pos)
            row = q_zn * Q_ROWS + c * SUB
            relay_rdmas.append(
                fwd(row, ys_sem.at[NQ + i], yr_sem.at[NQ + i], dev_y))
        for i, c in enumerate(Z_RELAY):
            pos = XQ_ORDER.index(c)
            y_rdmas[pos].wait_recv()
            y_recv_waited.add(pos)
            row = q_yn * Q_ROWS + c * SUB
            relay_rdmas.append(
                fwd(row, zs_sem.at[NQ + i], zr_sem.at[NQ + i], dev_z))

        for k in range(N_X):
            x_rdmas[k].wait_send()
            if k >= NQ:
                x_rdmas[k].wait_recv()
        for k in range(NQ):
            y_rdmas[k].wait_send()
            if k not in y_recv_waited:
                y_rdmas[k].wait_recv()
            z_rdmas[k].wait_send()
            if k not in z_recv_waited:
                z_rdmas[k].wait_recv()
        for r in relay_rdmas:
            r.wait()

    return pl.pallas_call(
        body,
        out_shape=jax.ShapeDtypeStruct(shard_shape, jnp.bfloat16),
        in_specs=[
            pl.BlockSpec(memory_space=pl.ANY),
            pl.BlockSpec(memory_space=pltpu.SMEM),
        ],
        out_specs=pl.BlockSpec(memory_space=pltpu.VMEM),
        scratch_shapes=[
            pltpu.VMEM((2, SUB, ncols), x.dtype),
            pltpu.VMEM((N_X * SUB, ncols), jnp.bfloat16),
            pltpu.SemaphoreType.DMA((2,)),
            pltpu.SemaphoreType.DMA((N_X,)),
            pltpu.SemaphoreType.DMA((N_X,)),
            pltpu.SemaphoreType.DMA((NQ + len(Y_RELAY),)),
            pltpu.SemaphoreType.DMA((NQ + len(Y_RELAY),)),
            pltpu.SemaphoreType.DMA((NQ + len(Z_RELAY),)),
            pltpu.SemaphoreType.DMA((NQ + len(Z_RELAY),)),
        ],
        compiler_params=pltpu.CompilerParams(collective_id=0),
    )(x, pi)
